# baseline (speedup 1.0000x reference)
import functools

import jax
import jax.numpy as jnp
from jax import lax
from jax.experimental import pallas as pl
from jax.experimental.pallas import tpu as pltpu
from jax.experimental.pallas import tpu_sc as plsc

B = 16384
C = 1000
CPAD = 1024
ROW_BLOCK = 1024
NS = 16
NW = 2 * NS
L = 16

R_SC = 4096
R_TC = B - R_SC
NB_TC = R_TC // ROW_BLOCK
ROWS_PER_TILE = R_SC // NW
CHUNK_ROWS = 16
N_CHUNKS = ROWS_PER_TILE // CHUNK_ROWS
NVEC = C // L
TAIL = C - NVEC * L

CHUNK = B // NS


def _ent_of(x):
    m = jnp.max(x, axis=1, keepdims=True)
    e = jnp.exp(x - m)
    s = jnp.sum(e, axis=1)
    u = jnp.sum(e * x, axis=1)
    return m[:, 0] + jnp.log(s) - u / s


def _entropy_body(x_ref, out_ref):
    out_ref[...] = _ent_of(x_ref[...]).reshape(1, 1, ROW_BLOCK)


def _rowwise_entropy_tc(prediction):
    ent = pl.pallas_call(
        _entropy_body,
        grid=(NB_TC,),
        in_specs=[pl.BlockSpec((ROW_BLOCK, C), lambda i: (i, 0))],
        out_specs=pl.BlockSpec((1, 1, ROW_BLOCK), lambda i: (i, 0, 0)),
        out_shape=jax.ShapeDtypeStruct((NB_TC, 1, ROW_BLOCK), jnp.float32),
    )(prediction)
    return ent.reshape(R_TC)


def _stats_body(pred_hbm, m_out, s_out, u_out, chunk_v, m_v, s_v, u_v):
    cc = lax.axis_index("c")
    ss = lax.axis_index("s")
    wid = ss * 2 + cc
    tile_base = R_TC + wid * ROWS_PER_TILE

    neg_big = jnp.float32(-3.0e38)
    tail_mask = lax.iota(jnp.int32, L) >= (L - TAIL)
    tail_off = (NVEC - 1) * L + TAIL

    lane = lax.iota(jnp.int32, L)
    zeros16 = jnp.zeros((L,), jnp.float32)

    def chunk_body(ci, _):
        row0 = tile_base + ci * CHUNK_ROWS
        pltpu.sync_copy(pred_hbm.at[pl.ds(row0, CHUNK_ROWS), :], chunk_v)

        def row_body(r, carry):
            mvec, svec, uvec = carry

            def max_body(k, m16):
                return jnp.maximum(m16, chunk_v[r, pl.ds(k * L, L)])
            m16 = lax.fori_loop(0, NVEC, max_body, jnp.full((L,), neg_big))
            t = chunk_v[r, pl.ds(tail_off, L)]
            m16 = jnp.maximum(m16, jnp.where(tail_mask, t, neg_big))
            m = jnp.max(m16)
            mb = jnp.full((L,), m)

            def sum_body(k, carry2):
                s16, u16 = carry2
                x = chunk_v[r, pl.ds(k * L, L)]
                e = jnp.exp(x - mb)
                return (s16 + e, u16 + e * x)
            s16, u16 = lax.fori_loop(0, NVEC, sum_body, (zeros16, zeros16))
            xt = chunk_v[r, pl.ds(tail_off, L)]
            et = jnp.where(tail_mask, jnp.exp(xt - mb), 0.0)
            s16 = s16 + et
            u16 = u16 + et * xt

            here = lane == r
            mvec = jnp.where(here, mb, mvec)
            svec = jnp.where(here, jnp.full((L,), jnp.sum(s16)), svec)
            uvec = jnp.where(here, jnp.full((L,), jnp.sum(u16)), uvec)
            return (mvec, svec, uvec)

        mvec, svec, uvec = lax.fori_loop(
            0, CHUNK_ROWS, row_body, (zeros16, zeros16, zeros16))
        m_v[pl.ds(ci * CHUNK_ROWS, CHUNK_ROWS)] = mvec
        s_v[pl.ds(ci * CHUNK_ROWS, CHUNK_ROWS)] = svec
        u_v[pl.ds(ci * CHUNK_ROWS, CHUNK_ROWS)] = uvec
        return 0
    lax.fori_loop(0, N_CHUNKS, chunk_body, 0)

    out_base = wid * ROWS_PER_TILE
    pltpu.sync_copy(m_v, m_out.at[pl.ds(out_base, ROWS_PER_TILE)])
    pltpu.sync_copy(s_v, s_out.at[pl.ds(out_base, ROWS_PER_TILE)])
    pltpu.sync_copy(u_v, u_out.at[pl.ds(out_base, ROWS_PER_TILE)])


@functools.cache
def _stats_call():
    return pl.kernel(
        _stats_body,
        out_type=(
            jax.ShapeDtypeStruct((R_SC,), jnp.float32),
            jax.ShapeDtypeStruct((R_SC,), jnp.float32),
            jax.ShapeDtypeStruct((R_SC,), jnp.float32),
        ),
        mesh=plsc.VectorSubcoreMesh(core_axis_name="c", subcore_axis_name="s"),
        compiler_params=pltpu.CompilerParams(needs_layout_passes=False),
        scratch_types=[
            pltpu.VMEM((CHUNK_ROWS, C), jnp.float32),
            pltpu.VMEM((ROWS_PER_TILE,), jnp.float32),
            pltpu.VMEM((ROWS_PER_TILE,), jnp.float32),
            pltpu.VMEM((ROWS_PER_TILE,), jnp.float32),
        ],
    )


def _epilogue_body(m_ref, s_ref, u_ref, out_ref):
    m = m_ref[...]
    s = s_ref[...]
    u = u_ref[...]
    out_ref[...] = m + jnp.log(s) - u / s


def _entropy_sc(prediction):
    m, s, u = _stats_call()(prediction)
    return pl.pallas_call(
        _epilogue_body,
        out_shape=jax.ShapeDtypeStruct((R_SC,), jnp.float32),
    )(m, s, u)


def _hist_body(ent_hbm, tgt_hbm, out_hbm,
               tgt_v, val_v, hist_v, part_v, out_v, shared):
    c = lax.axis_index("c")
    s = lax.axis_index("s")
    base = s * CHUNK

    zeros16 = jnp.zeros((L,), jnp.float32)

    def zero_body(i, _):
        hist_v[pl.ds(i * L, L)] = zeros16
        return 0
    lax.fori_loop(0, CPAD // L, zero_body, 0)

    pltpu.sync_copy(tgt_hbm.at[pl.ds(base, CHUNK)], tgt_v)

    @pl.when(c == 0)
    def _():
        pltpu.sync_copy(ent_hbm.at[pl.ds(base, CHUNK)], val_v)

    @pl.when(c != 0)
    def _():
        ones16 = jnp.ones((L,), jnp.float32)
        def ones_body(i, _):
            val_v[pl.ds(i * L, L)] = ones16
            return 0
        lax.fori_loop(0, CHUNK // L, ones_body, 0)

    def scat_body(j, _):
        idx = tgt_v[pl.ds(j * L, L)]
        val = val_v[pl.ds(j * L, L)]
        plsc.addupdate_scatter(hist_v, [idx], val)
        return 0
    lax.fori_loop(0, CHUNK // L, scat_body, 0)

    pltpu.sync_copy(hist_v, shared.at[s])
    plsc.subcore_barrier()
    pltpu.sync_copy(shared, part_v)

    span = CPAD // NS
    for k in range(span // L):
        acc = zeros16
        for r in range(NS):
            acc = acc + part_v[r, pl.ds(s * span + k * L, L)]
        out_v[pl.ds(k * L, L)] = acc

    pltpu.sync_copy(out_v, out_hbm.at[pl.ds(c * CPAD + s * span, span)])


@functools.cache
def _hist_call():
    return pl.kernel(
        _hist_body,
        out_type=jax.ShapeDtypeStruct((2 * CPAD,), jnp.float32),
        mesh=plsc.VectorSubcoreMesh(core_axis_name="c", subcore_axis_name="s"),
        compiler_params=pltpu.CompilerParams(needs_layout_passes=False),
        scratch_types=[
            pltpu.VMEM((CHUNK,), jnp.int32),
            pltpu.VMEM((CHUNK,), jnp.float32),
            pltpu.VMEM((CPAD,), jnp.float32),
            pltpu.VMEM((NS, CPAD), jnp.float32),
            pltpu.VMEM((CPAD // NS,), jnp.float32),
            pltpu.VMEM_SHARED((NS, CPAD), jnp.float32),
        ],
    )


def kernel(prediction, target):
    ent_tc = _rowwise_entropy_tc(prediction)
    ent_sc = _entropy_sc(prediction)
    ent = jnp.concatenate([ent_tc, ent_sc])
    tgt = target.astype(jnp.int32)
    out = _hist_call()(ent, tgt)
    return out[:C], out[CPAD:CPAD + C]


# speedup vs baseline: 1.3890x; 1.3890x over previous
import functools

import jax
import jax.numpy as jnp
from jax import lax
from jax.experimental import pallas as pl
from jax.experimental.pallas import tpu as pltpu
from jax.experimental.pallas import tpu_sc as plsc

B = 16384
C = 1000
CPAD = 1024
ROW_BLOCK = 1024
NS = 16
NW = 2 * NS
L = 16

R_SC = 4096
R_TC = B - R_SC
NB_TC = R_TC // ROW_BLOCK
ROWS_PER_TILE = R_SC // NW
CHUNK_ROWS = 16
N_CHUNKS = ROWS_PER_TILE // CHUNK_ROWS
NVEC = C // L
TAIL = C - NVEC * L

CHUNK = B // NS


def _ent_of(x):
    m = jnp.max(x, axis=1, keepdims=True)
    e = jnp.exp(x - m)
    s = jnp.sum(e, axis=1)
    u = jnp.sum(e * x, axis=1)
    return m[:, 0] + jnp.log(s) - u / s


def _entropy_body(x_ref, out_ref):
    out_ref[...] = _ent_of(x_ref[...]).reshape(1, 1, ROW_BLOCK)


def _rowwise_entropy_tc(prediction):
    ent = pl.pallas_call(
        _entropy_body,
        grid=(NB_TC,),
        in_specs=[pl.BlockSpec((ROW_BLOCK, C), lambda i: (i, 0))],
        out_specs=pl.BlockSpec((1, 1, ROW_BLOCK), lambda i: (i, 0, 0)),
        out_shape=jax.ShapeDtypeStruct((NB_TC, 1, ROW_BLOCK), jnp.float32),
    )(prediction)
    return ent.reshape(R_TC)


def _stats_body(pred_hbm, m_out, s_out, u_out, chunk_v, m_v, s_v, u_v):
    cc = lax.axis_index("c")
    ss = lax.axis_index("s")
    wid = ss * 2 + cc
    tile_base = R_TC + wid * ROWS_PER_TILE

    neg_big = jnp.float32(-3.0e38)
    tail_mask = lax.iota(jnp.int32, L) >= (L - TAIL)
    tail_off = (NVEC - 1) * L + TAIL

    lane = lax.iota(jnp.int32, L)
    zeros16 = jnp.zeros((L,), jnp.float32)

    def chunk_body(ci, _):
        row0 = tile_base + ci * CHUNK_ROWS
        pltpu.sync_copy(pred_hbm.at[pl.ds(row0, CHUNK_ROWS), :], chunk_v)

        def row_body(r, carry):
            mvec, svec, uvec = carry

            m16 = chunk_v[r, pl.ds(0, L)]
            for k in range(1, NVEC):
                m16 = jnp.maximum(m16, chunk_v[r, pl.ds(k * L, L)])
            xt = chunk_v[r, pl.ds(tail_off, L)]
            m16 = jnp.maximum(m16, jnp.where(tail_mask, xt, neg_big))
            m = jnp.max(m16)
            mb = jnp.full((L,), m)

            s16 = zeros16
            u16 = zeros16
            for k in range(NVEC):
                x = chunk_v[r, pl.ds(k * L, L)]
                e = jnp.exp(x - mb)
                s16 = s16 + e
                u16 = u16 + e * x
            et = jnp.where(tail_mask, jnp.exp(xt - mb), 0.0)
            s16 = s16 + et
            u16 = u16 + et * xt

            here = lane == r
            mvec = jnp.where(here, mb, mvec)
            svec = jnp.where(here, jnp.full((L,), jnp.sum(s16)), svec)
            uvec = jnp.where(here, jnp.full((L,), jnp.sum(u16)), uvec)
            return (mvec, svec, uvec)

        mvec, svec, uvec = lax.fori_loop(
            0, CHUNK_ROWS, row_body, (zeros16, zeros16, zeros16))
        m_v[pl.ds(ci * CHUNK_ROWS, CHUNK_ROWS)] = mvec
        s_v[pl.ds(ci * CHUNK_ROWS, CHUNK_ROWS)] = svec
        u_v[pl.ds(ci * CHUNK_ROWS, CHUNK_ROWS)] = uvec
        return 0
    lax.fori_loop(0, N_CHUNKS, chunk_body, 0)

    out_base = wid * ROWS_PER_TILE
    pltpu.sync_copy(m_v, m_out.at[pl.ds(out_base, ROWS_PER_TILE)])
    pltpu.sync_copy(s_v, s_out.at[pl.ds(out_base, ROWS_PER_TILE)])
    pltpu.sync_copy(u_v, u_out.at[pl.ds(out_base, ROWS_PER_TILE)])


@functools.cache
def _stats_call():
    return pl.kernel(
        _stats_body,
        out_type=(
            jax.ShapeDtypeStruct((R_SC,), jnp.float32),
            jax.ShapeDtypeStruct((R_SC,), jnp.float32),
            jax.ShapeDtypeStruct((R_SC,), jnp.float32),
        ),
        mesh=plsc.VectorSubcoreMesh(core_axis_name="c", subcore_axis_name="s"),
        compiler_params=pltpu.CompilerParams(needs_layout_passes=False),
        scratch_types=[
            pltpu.VMEM((CHUNK_ROWS, C), jnp.float32),
            pltpu.VMEM((ROWS_PER_TILE,), jnp.float32),
            pltpu.VMEM((ROWS_PER_TILE,), jnp.float32),
            pltpu.VMEM((ROWS_PER_TILE,), jnp.float32),
        ],
    )


def _epilogue_body(m_ref, s_ref, u_ref, out_ref):
    m = m_ref[...]
    s = s_ref[...]
    u = u_ref[...]
    out_ref[...] = m + jnp.log(s) - u / s


def _entropy_sc(prediction):
    m, s, u = _stats_call()(prediction)
    return pl.pallas_call(
        _epilogue_body,
        out_shape=jax.ShapeDtypeStruct((R_SC,), jnp.float32),
    )(m, s, u)


def _hist_body(ent_hbm, tgt_hbm, out_hbm,
               tgt_v, val_v, hist_v, part_v, out_v, shared):
    c = lax.axis_index("c")
    s = lax.axis_index("s")
    base = s * CHUNK

    zeros16 = jnp.zeros((L,), jnp.float32)

    def zero_body(i, _):
        hist_v[pl.ds(i * L, L)] = zeros16
        return 0
    lax.fori_loop(0, CPAD // L, zero_body, 0)

    pltpu.sync_copy(tgt_hbm.at[pl.ds(base, CHUNK)], tgt_v)

    @pl.when(c == 0)
    def _():
        pltpu.sync_copy(ent_hbm.at[pl.ds(base, CHUNK)], val_v)

    @pl.when(c != 0)
    def _():
        ones16 = jnp.ones((L,), jnp.float32)
        def ones_body(i, _):
            val_v[pl.ds(i * L, L)] = ones16
            return 0
        lax.fori_loop(0, CHUNK // L, ones_body, 0)

    def scat_body(j, _):
        idx = tgt_v[pl.ds(j * L, L)]
        val = val_v[pl.ds(j * L, L)]
        plsc.addupdate_scatter(hist_v, [idx], val)
        return 0
    lax.fori_loop(0, CHUNK // L, scat_body, 0)

    pltpu.sync_copy(hist_v, shared.at[s])
    plsc.subcore_barrier()
    pltpu.sync_copy(shared, part_v)

    span = CPAD // NS
    for k in range(span // L):
        acc = zeros16
        for r in range(NS):
            acc = acc + part_v[r, pl.ds(s * span + k * L, L)]
        out_v[pl.ds(k * L, L)] = acc

    pltpu.sync_copy(out_v, out_hbm.at[pl.ds(c * CPAD + s * span, span)])


@functools.cache
def _hist_call():
    return pl.kernel(
        _hist_body,
        out_type=jax.ShapeDtypeStruct((2 * CPAD,), jnp.float32),
        mesh=plsc.VectorSubcoreMesh(core_axis_name="c", subcore_axis_name="s"),
        compiler_params=pltpu.CompilerParams(needs_layout_passes=False),
        scratch_types=[
            pltpu.VMEM((CHUNK,), jnp.int32),
            pltpu.VMEM((CHUNK,), jnp.float32),
            pltpu.VMEM((CPAD,), jnp.float32),
            pltpu.VMEM((NS, CPAD), jnp.float32),
            pltpu.VMEM((CPAD // NS,), jnp.float32),
            pltpu.VMEM_SHARED((NS, CPAD), jnp.float32),
        ],
    )


def kernel(prediction, target):
    ent_tc = _rowwise_entropy_tc(prediction)
    ent_sc = _entropy_sc(prediction)
    ent = jnp.concatenate([ent_tc, ent_sc])
    tgt = target.astype(jnp.int32)
    out = _hist_call()(ent, tgt)
    return out[:C], out[CPAD:CPAD + C]


# speedup vs baseline: 1.3935x; 1.0032x over previous
import functools

import jax
import jax.numpy as jnp
from jax import lax
from jax.experimental import pallas as pl
from jax.experimental.pallas import tpu as pltpu
from jax.experimental.pallas import tpu_sc as plsc

B = 16384
C = 1000
CPAD = 1024
ROW_BLOCK = 1024
NS = 16
NW = 2 * NS
L = 16

R_SC = 4096
R_TC = B - R_SC
NB_TC = R_TC // ROW_BLOCK
ROWS_PER_TILE = R_SC // NW
CHUNK_ROWS = 16
N_CHUNKS = ROWS_PER_TILE // CHUNK_ROWS
NVEC = C // L
TAIL = C - NVEC * L

CHUNK = B // NS


def _ent_of(x):
    m = jnp.max(x, axis=1, keepdims=True)
    e = jnp.exp(x - m)
    s = jnp.sum(e, axis=1)
    u = jnp.sum(e * x, axis=1)
    return m[:, 0] + jnp.log(s) - u / s


def _entropy_body(x_ref, out_ref):
    out_ref[...] = _ent_of(x_ref[...]).reshape(1, 1, ROW_BLOCK)


def _rowwise_entropy_tc(prediction):
    ent = pl.pallas_call(
        _entropy_body,
        grid=(NB_TC,),
        in_specs=[pl.BlockSpec((ROW_BLOCK, C), lambda i: (i, 0))],
        out_specs=pl.BlockSpec((1, 1, ROW_BLOCK), lambda i: (i, 0, 0)),
        out_shape=jax.ShapeDtypeStruct((NB_TC, 1, ROW_BLOCK), jnp.float32),
    )(prediction)
    return ent.reshape(R_TC)


def _stats_body(pred_hbm, m_out, s_out, u_out, chunk_v, m_v, s_v, u_v):
    cc = lax.axis_index("c")
    ss = lax.axis_index("s")
    wid = ss * 2 + cc
    tile_base = R_TC + wid * ROWS_PER_TILE

    neg_big = jnp.float32(-3.0e38)
    tail_mask = lax.iota(jnp.int32, L) >= (L - TAIL)
    tail_off = (NVEC - 1) * L + TAIL

    lane = lax.iota(jnp.int32, L)
    zeros16 = jnp.zeros((L,), jnp.float32)

    def chunk_body(ci, _):
        row0 = tile_base + ci * CHUNK_ROWS
        pltpu.sync_copy(pred_hbm.at[pl.ds(row0, CHUNK_ROWS), :], chunk_v)

        def row_body(r, carry):
            mvec, svec, uvec = carry

            m16 = chunk_v[r, pl.ds(0, L)]
            for k in range(1, NVEC):
                m16 = jnp.maximum(m16, chunk_v[r, pl.ds(k * L, L)])
            xt = chunk_v[r, pl.ds(tail_off, L)]
            m16 = jnp.maximum(m16, jnp.where(tail_mask, xt, neg_big))
            m = jnp.max(m16)
            mb = jnp.full((L,), m)

            s16 = zeros16
            u16 = zeros16
            for k in range(NVEC):
                x = chunk_v[r, pl.ds(k * L, L)]
                e = jnp.exp(x - mb)
                s16 = s16 + e
                u16 = u16 + e * x
            et = jnp.where(tail_mask, jnp.exp(xt - mb), 0.0)
            s16 = s16 + et
            u16 = u16 + et * xt

            here = lane == r
            mvec = jnp.where(here, mb, mvec)
            svec = jnp.where(here, jnp.full((L,), jnp.sum(s16)), svec)
            uvec = jnp.where(here, jnp.full((L,), jnp.sum(u16)), uvec)
            return (mvec, svec, uvec)

        mvec, svec, uvec = lax.fori_loop(
            0, CHUNK_ROWS, row_body, (zeros16, zeros16, zeros16))
        m_v[pl.ds(ci * CHUNK_ROWS, CHUNK_ROWS)] = mvec
        s_v[pl.ds(ci * CHUNK_ROWS, CHUNK_ROWS)] = svec
        u_v[pl.ds(ci * CHUNK_ROWS, CHUNK_ROWS)] = uvec
        return 0
    lax.fori_loop(0, N_CHUNKS, chunk_body, 0)

    out_base = wid * ROWS_PER_TILE
    pltpu.sync_copy(m_v, m_out.at[pl.ds(out_base, ROWS_PER_TILE)])
    pltpu.sync_copy(s_v, s_out.at[pl.ds(out_base, ROWS_PER_TILE)])
    pltpu.sync_copy(u_v, u_out.at[pl.ds(out_base, ROWS_PER_TILE)])


@functools.cache
def _stats_call():
    return pl.kernel(
        _stats_body,
        out_type=(
            jax.ShapeDtypeStruct((R_SC,), jnp.float32),
            jax.ShapeDtypeStruct((R_SC,), jnp.float32),
            jax.ShapeDtypeStruct((R_SC,), jnp.float32),
        ),
        mesh=plsc.VectorSubcoreMesh(core_axis_name="c", subcore_axis_name="s"),
        compiler_params=pltpu.CompilerParams(needs_layout_passes=False),
        cost_estimate=pl.CostEstimate(
            flops=5 * R_SC * C,
            bytes_accessed=4 * R_SC * C,
            transcendentals=R_SC * C,
        ),
        scratch_types=[
            pltpu.VMEM((CHUNK_ROWS, C), jnp.float32),
            pltpu.VMEM((ROWS_PER_TILE,), jnp.float32),
            pltpu.VMEM((ROWS_PER_TILE,), jnp.float32),
            pltpu.VMEM((ROWS_PER_TILE,), jnp.float32),
        ],
    )


def _epilogue_body(m_ref, s_ref, u_ref, out_ref):
    m = m_ref[...]
    s = s_ref[...]
    u = u_ref[...]
    out_ref[...] = m + jnp.log(s) - u / s


def _entropy_sc(prediction):
    m, s, u = _stats_call()(prediction)
    return pl.pallas_call(
        _epilogue_body,
        out_shape=jax.ShapeDtypeStruct((R_SC,), jnp.float32),
    )(m, s, u)


def _hist_body(ent_hbm, tgt_hbm, out_hbm,
               tgt_v, val_v, hist_v, part_v, out_v, shared):
    c = lax.axis_index("c")
    s = lax.axis_index("s")
    base = s * CHUNK

    zeros16 = jnp.zeros((L,), jnp.float32)

    def zero_body(i, _):
        hist_v[pl.ds(i * L, L)] = zeros16
        return 0
    lax.fori_loop(0, CPAD // L, zero_body, 0)

    pltpu.sync_copy(tgt_hbm.at[pl.ds(base, CHUNK)], tgt_v)

    @pl.when(c == 0)
    def _():
        pltpu.sync_copy(ent_hbm.at[pl.ds(base, CHUNK)], val_v)

    @pl.when(c != 0)
    def _():
        ones16 = jnp.ones((L,), jnp.float32)
        def ones_body(i, _):
            val_v[pl.ds(i * L, L)] = ones16
            return 0
        lax.fori_loop(0, CHUNK // L, ones_body, 0)

    def scat_body(j, _):
        idx = tgt_v[pl.ds(j * L, L)]
        val = val_v[pl.ds(j * L, L)]
        plsc.addupdate_scatter(hist_v, [idx], val)
        return 0
    lax.fori_loop(0, CHUNK // L, scat_body, 0)

    pltpu.sync_copy(hist_v, shared.at[s])
    plsc.subcore_barrier()
    pltpu.sync_copy(shared, part_v)

    span = CPAD // NS
    for k in range(span // L):
        acc = zeros16
        for r in range(NS):
            acc = acc + part_v[r, pl.ds(s * span + k * L, L)]
        out_v[pl.ds(k * L, L)] = acc

    pltpu.sync_copy(out_v, out_hbm.at[pl.ds(c * CPAD + s * span, span)])


@functools.cache
def _hist_call():
    return pl.kernel(
        _hist_body,
        out_type=jax.ShapeDtypeStruct((2 * CPAD,), jnp.float32),
        mesh=plsc.VectorSubcoreMesh(core_axis_name="c", subcore_axis_name="s"),
        compiler_params=pltpu.CompilerParams(needs_layout_passes=False),
        scratch_types=[
            pltpu.VMEM((CHUNK,), jnp.int32),
            pltpu.VMEM((CHUNK,), jnp.float32),
            pltpu.VMEM((CPAD,), jnp.float32),
            pltpu.VMEM((NS, CPAD), jnp.float32),
            pltpu.VMEM((CPAD // NS,), jnp.float32),
            pltpu.VMEM_SHARED((NS, CPAD), jnp.float32),
        ],
    )


def kernel(prediction, target):
    ent_tc = _rowwise_entropy_tc(prediction)
    ent_sc = _entropy_sc(prediction)
    ent = jnp.concatenate([ent_tc, ent_sc])
    tgt = target.astype(jnp.int32)
    out = _hist_call()(ent, tgt)
    return out[:C], out[CPAD:CPAD + C]


# speedup vs baseline: 1.3955x; 1.0015x over previous
import functools

import jax
import jax.numpy as jnp
from jax import lax
from jax.experimental import pallas as pl
from jax.experimental.pallas import tpu as pltpu
from jax.experimental.pallas import tpu_sc as plsc

B = 16384
C = 1000
CPAD = 1024
ROW_BLOCK = 1024
NS = 16
NW = 2 * NS
L = 16

R_SC = 4096
R_TC = B - R_SC
NB_TC = R_TC // ROW_BLOCK
ROWS_PER_TILE = R_SC // NW
CHUNK_ROWS = 16
N_CHUNKS = ROWS_PER_TILE // CHUNK_ROWS
NVEC = C // L
TAIL = C - NVEC * L

CHUNK = B // NS


def _ent_of(x):
    m = jnp.max(x, axis=1, keepdims=True)
    e = jnp.exp(x - m)
    s = jnp.sum(e, axis=1)
    u = jnp.sum(e * x, axis=1)
    return m[:, 0] + jnp.log(s) - u / s


def _entropy_body(x_ref, out_ref):
    out_ref[...] = _ent_of(x_ref[...]).reshape(1, 1, ROW_BLOCK)


def _rowwise_entropy_tc(prediction):
    ent = pl.pallas_call(
        _entropy_body,
        grid=(NB_TC,),
        in_specs=[pl.BlockSpec((ROW_BLOCK, C), lambda i: (i, 0))],
        out_specs=pl.BlockSpec((1, 1, ROW_BLOCK), lambda i: (i, 0, 0)),
        out_shape=jax.ShapeDtypeStruct((NB_TC, 1, ROW_BLOCK), jnp.float32),
        compiler_params=pltpu.CompilerParams(skip_device_barrier=True),
    )(prediction)
    return ent.reshape(R_TC)


def _stats_body(pred_hbm, m_out, s_out, u_out, chunk_v, m_v, s_v, u_v):
    cc = lax.axis_index("c")
    ss = lax.axis_index("s")
    wid = ss * 2 + cc
    tile_base = R_TC + wid * ROWS_PER_TILE

    neg_big = jnp.float32(-3.0e38)
    tail_mask = lax.iota(jnp.int32, L) >= (L - TAIL)
    tail_off = (NVEC - 1) * L + TAIL

    lane = lax.iota(jnp.int32, L)
    zeros16 = jnp.zeros((L,), jnp.float32)

    def chunk_body(ci, _):
        row0 = tile_base + ci * CHUNK_ROWS
        pltpu.sync_copy(pred_hbm.at[pl.ds(row0, CHUNK_ROWS), :], chunk_v)

        def row_body(r, carry):
            mvec, svec, uvec = carry

            m16 = chunk_v[r, pl.ds(0, L)]
            for k in range(1, NVEC):
                m16 = jnp.maximum(m16, chunk_v[r, pl.ds(k * L, L)])
            xt = chunk_v[r, pl.ds(tail_off, L)]
            m16 = jnp.maximum(m16, jnp.where(tail_mask, xt, neg_big))
            m = jnp.max(m16)
            mb = jnp.full((L,), m)

            s16 = zeros16
            u16 = zeros16
            for k in range(NVEC):
                x = chunk_v[r, pl.ds(k * L, L)]
                e = jnp.exp(x - mb)
                s16 = s16 + e
                u16 = u16 + e * x
            et = jnp.where(tail_mask, jnp.exp(xt - mb), 0.0)
            s16 = s16 + et
            u16 = u16 + et * xt

            here = lane == r
            mvec = jnp.where(here, mb, mvec)
            svec = jnp.where(here, jnp.full((L,), jnp.sum(s16)), svec)
            uvec = jnp.where(here, jnp.full((L,), jnp.sum(u16)), uvec)
            return (mvec, svec, uvec)

        mvec, svec, uvec = lax.fori_loop(
            0, CHUNK_ROWS, row_body, (zeros16, zeros16, zeros16))
        m_v[pl.ds(ci * CHUNK_ROWS, CHUNK_ROWS)] = mvec
        s_v[pl.ds(ci * CHUNK_ROWS, CHUNK_ROWS)] = svec
        u_v[pl.ds(ci * CHUNK_ROWS, CHUNK_ROWS)] = uvec
        return 0
    lax.fori_loop(0, N_CHUNKS, chunk_body, 0)

    out_base = wid * ROWS_PER_TILE
    pltpu.sync_copy(m_v, m_out.at[pl.ds(out_base, ROWS_PER_TILE)])
    pltpu.sync_copy(s_v, s_out.at[pl.ds(out_base, ROWS_PER_TILE)])
    pltpu.sync_copy(u_v, u_out.at[pl.ds(out_base, ROWS_PER_TILE)])


@functools.cache
def _stats_call():
    return pl.kernel(
        _stats_body,
        out_type=(
            jax.ShapeDtypeStruct((R_SC,), jnp.float32),
            jax.ShapeDtypeStruct((R_SC,), jnp.float32),
            jax.ShapeDtypeStruct((R_SC,), jnp.float32),
        ),
        mesh=plsc.VectorSubcoreMesh(core_axis_name="c", subcore_axis_name="s"),
        compiler_params=pltpu.CompilerParams(
            needs_layout_passes=False, skip_device_barrier=True),
        cost_estimate=pl.CostEstimate(
            flops=5 * R_SC * C,
            bytes_accessed=4 * R_SC * C,
            transcendentals=R_SC * C,
        ),
        scratch_types=[
            pltpu.VMEM((CHUNK_ROWS, C), jnp.float32),
            pltpu.VMEM((ROWS_PER_TILE,), jnp.float32),
            pltpu.VMEM((ROWS_PER_TILE,), jnp.float32),
            pltpu.VMEM((ROWS_PER_TILE,), jnp.float32),
        ],
    )


def _epilogue_body(m_ref, s_ref, u_ref, out_ref):
    m = m_ref[...]
    s = s_ref[...]
    u = u_ref[...]
    out_ref[...] = m + jnp.log(s) - u / s


def _entropy_sc(prediction):
    m, s, u = _stats_call()(prediction)
    return pl.pallas_call(
        _epilogue_body,
        out_shape=jax.ShapeDtypeStruct((R_SC,), jnp.float32),
    )(m, s, u)


def _hist_body(ent_hbm, tgt_hbm, out_hbm,
               tgt_v, val_v, hist_v, part_v, out_v, shared):
    c = lax.axis_index("c")
    s = lax.axis_index("s")
    base = s * CHUNK

    zeros16 = jnp.zeros((L,), jnp.float32)

    def zero_body(i, _):
        hist_v[pl.ds(i * L, L)] = zeros16
        return 0
    lax.fori_loop(0, CPAD // L, zero_body, 0)

    pltpu.sync_copy(tgt_hbm.at[pl.ds(base, CHUNK)], tgt_v)

    @pl.when(c == 0)
    def _():
        pltpu.sync_copy(ent_hbm.at[pl.ds(base, CHUNK)], val_v)

    @pl.when(c != 0)
    def _():
        ones16 = jnp.ones((L,), jnp.float32)
        def ones_body(i, _):
            val_v[pl.ds(i * L, L)] = ones16
            return 0
        lax.fori_loop(0, CHUNK // L, ones_body, 0)

    def scat_body(j, _):
        idx = tgt_v[pl.ds(j * L, L)]
        val = val_v[pl.ds(j * L, L)]
        plsc.addupdate_scatter(hist_v, [idx], val)
        return 0
    lax.fori_loop(0, CHUNK // L, scat_body, 0)

    pltpu.sync_copy(hist_v, shared.at[s])
    plsc.subcore_barrier()
    pltpu.sync_copy(shared, part_v)

    span = CPAD // NS
    for k in range(span // L):
        acc = zeros16
        for r in range(NS):
            acc = acc + part_v[r, pl.ds(s * span + k * L, L)]
        out_v[pl.ds(k * L, L)] = acc

    pltpu.sync_copy(out_v, out_hbm.at[pl.ds(c * CPAD + s * span, span)])


@functools.cache
def _hist_call():
    return pl.kernel(
        _hist_body,
        out_type=jax.ShapeDtypeStruct((2 * CPAD,), jnp.float32),
        mesh=plsc.VectorSubcoreMesh(core_axis_name="c", subcore_axis_name="s"),
        compiler_params=pltpu.CompilerParams(needs_layout_passes=False),
        scratch_types=[
            pltpu.VMEM((CHUNK,), jnp.int32),
            pltpu.VMEM((CHUNK,), jnp.float32),
            pltpu.VMEM((CPAD,), jnp.float32),
            pltpu.VMEM((NS, CPAD), jnp.float32),
            pltpu.VMEM((CPAD // NS,), jnp.float32),
            pltpu.VMEM_SHARED((NS, CPAD), jnp.float32),
        ],
    )


def kernel(prediction, target):
    ent_tc = _rowwise_entropy_tc(prediction)
    ent_sc = _entropy_sc(prediction)
    ent = jnp.concatenate([ent_tc, ent_sc])
    tgt = target.astype(jnp.int32)
    out = _hist_call()(ent, tgt)
    return out[:C], out[CPAD:CPAD + C]


# speedup vs baseline: 1.5127x; 1.0839x over previous
import functools

import jax
import jax.numpy as jnp
from jax import lax
from jax.experimental import pallas as pl
from jax.experimental.pallas import tpu as pltpu
from jax.experimental.pallas import tpu_sc as plsc

B = 16384
C = 1000
CPAD = 1024
ROW_BLOCK = 2048
NB = B // ROW_BLOCK
NS = 16
CHUNK = B // NS
L = 16


def _entropy_body(x_ref, out_ref):
    x = x_ref[...]
    m = jnp.max(x, axis=1, keepdims=True)
    e = jnp.exp(x - m)
    s = jnp.sum(e, axis=1)
    u = jnp.sum(e * x, axis=1)
    ent = m[:, 0] + jnp.log(s) - u / s
    out_ref[...] = ent.reshape(1, 1, ROW_BLOCK)


def _rowwise_entropy(prediction):
    ent = pl.pallas_call(
        _entropy_body,
        grid=(NB,),
        in_specs=[pl.BlockSpec((ROW_BLOCK, C), lambda i: (i, 0))],
        out_specs=pl.BlockSpec((1, 1, ROW_BLOCK), lambda i: (i, 0, 0)),
        out_shape=jax.ShapeDtypeStruct((NB, 1, ROW_BLOCK), jnp.float32),
    )(prediction)
    return ent.reshape(B)


def _hist_body(ent_hbm, tgt_hbm, out_hbm,
               tgt_v, val_v, hist_v, part_v, out_v, shared):
    c = lax.axis_index("c")
    s = lax.axis_index("s")
    base = s * CHUNK

    zeros16 = jnp.zeros((L,), jnp.float32)

    def zero_body(i, _):
        hist_v[pl.ds(i * L, L)] = zeros16
        return 0
    lax.fori_loop(0, CPAD // L, zero_body, 0)

    pltpu.sync_copy(tgt_hbm.at[pl.ds(base, CHUNK)], tgt_v)

    @pl.when(c == 0)
    def _():
        pltpu.sync_copy(ent_hbm.at[pl.ds(base, CHUNK)], val_v)

    @pl.when(c != 0)
    def _():
        ones16 = jnp.ones((L,), jnp.float32)
        def ones_body(i, _):
            val_v[pl.ds(i * L, L)] = ones16
            return 0
        lax.fori_loop(0, CHUNK // L, ones_body, 0)

    def scat_body(j, _):
        idx = tgt_v[pl.ds(j * L, L)]
        val = val_v[pl.ds(j * L, L)]
        plsc.addupdate_scatter(hist_v, [idx], val)
        return 0
    lax.fori_loop(0, CHUNK // L, scat_body, 0)

    pltpu.sync_copy(hist_v, shared.at[s])
    plsc.subcore_barrier()
    pltpu.sync_copy(shared, part_v)

    span = CPAD // NS
    for k in range(span // L):
        acc = zeros16
        for r in range(NS):
            acc = acc + part_v[r, pl.ds(s * span + k * L, L)]
        out_v[pl.ds(k * L, L)] = acc

    pltpu.sync_copy(out_v, out_hbm.at[pl.ds(c * CPAD + s * span, span)])


@functools.cache
def _hist_call():
    return pl.kernel(
        _hist_body,
        out_type=jax.ShapeDtypeStruct((2 * CPAD,), jnp.float32),
        mesh=plsc.VectorSubcoreMesh(core_axis_name="c", subcore_axis_name="s"),
        compiler_params=pltpu.CompilerParams(needs_layout_passes=False),
        scratch_types=[
            pltpu.VMEM((CHUNK,), jnp.int32),
            pltpu.VMEM((CHUNK,), jnp.float32),
            pltpu.VMEM((CPAD,), jnp.float32),
            pltpu.VMEM((NS, CPAD), jnp.float32),
            pltpu.VMEM((CPAD // NS,), jnp.float32),
            pltpu.VMEM_SHARED((NS, CPAD), jnp.float32),
        ],
    )


def kernel(prediction, target):
    ent = _rowwise_entropy(prediction)
    tgt = target.astype(jnp.int32)
    out = _hist_call()(ent, tgt)
    return out[:C], out[CPAD:CPAD + C]
